# baseline (speedup 1.0000x reference)
import functools

import jax
import jax.numpy as jnp
from jax import lax
from jax.experimental import pallas as pl
from jax.experimental.pallas import tpu as pltpu
from jax.experimental.pallas import tpu_sc as plsc

_HID = 128
_CD = 16
_NORB = 16

_NC = 2
_NS = 16
_NW = _NC * _NS
_CHUNK = 128

_B_TC = 400


def _sc_gather(table, z_pad):
    b_pad = z_pad.shape[0]
    b_per_w = b_pad // _NW
    n_chunks = b_per_w // _CHUNK
    mesh = plsc.VectorSubcoreMesh(
        core_axis_name="c", subcore_axis_name="s",
        num_cores=_NC, num_subcores=_NS)

    def body(table_hbm, idx_hbm, out_hbm, idx_v, rows_v, sem):
        wid = lax.axis_index("s") * _NC + lax.axis_index("c")
        base = wid * b_per_w

        def step(j, carry):
            off = base + j * _CHUNK
            pltpu.sync_copy(idx_hbm.at[pl.ds(off, _CHUNK)], idx_v)
            pltpu.async_copy(table_hbm.at[idx_v], rows_v, sem).wait()
            pltpu.sync_copy(rows_v, out_hbm.at[pl.ds(off, _CHUNK)])
            return carry

        lax.fori_loop(0, n_chunks, step, 0)

    f = pl.kernel(
        body,
        out_type=jax.ShapeDtypeStruct((b_pad, _HID), jnp.float32),
        mesh=mesh,
        scratch_types=[
            pltpu.VMEM((_CHUNK,), jnp.int32),
            pltpu.VMEM((_CHUNK, _HID), jnp.float32),
            pltpu.SemaphoreType.DMA,
        ],
    )
    return f(table, z_pad)


def _tc_body(cf_ref, ze_ref, w1_ref, w2_ref, out_ref):
    c = cf_ref[:]
    c = c * jax.nn.sigmoid(c)
    h = jnp.dot(c, w1_ref[:], preferred_element_type=jnp.float32)
    h = h * jax.nn.sigmoid(h)
    s = jnp.sum(h.reshape(_B_TC, _NORB, _HID), axis=1)
    out_ref[:] = ze_ref[:] + jnp.dot(
        s, w2_ref[:], preferred_element_type=jnp.float32)


def _tc_mlp(ze_pad, coeffs2, W1, W2, n):
    nblk = n // _B_TC
    return pl.pallas_call(
        _tc_body,
        grid=(nblk,),
        in_specs=[
            pl.BlockSpec((_B_TC * _NORB, _CD), lambda i: (i, 0)),
            pl.BlockSpec((_B_TC, _HID), lambda i: (i, 0)),
            pl.BlockSpec((_CD, _HID), lambda i: (0, 0)),
            pl.BlockSpec((_HID, _HID), lambda i: (0, 0)),
        ],
        out_specs=pl.BlockSpec((_B_TC, _HID), lambda i: (i, 0)),
        out_shape=jax.ShapeDtypeStruct((n, _HID), jnp.float32),
    )(coeffs2, ze_pad, W1, W2)


def kernel(z, coeffs, table, W1, W2):
    n = z.shape[0]
    granule = _NW * _CHUNK
    b_pad = ((n + granule - 1) // granule) * granule
    z_pad = jnp.pad(z.astype(jnp.int32), (0, b_pad - n))
    ze_pad = _sc_gather(table, z_pad)
    coeffs2 = coeffs.reshape(n * _NORB, _CD)
    return _tc_mlp(ze_pad, coeffs2, W1, W2, n)


# speedup vs baseline: 1.0227x; 1.0227x over previous
import functools

import jax
import jax.numpy as jnp
from jax import lax
from jax.experimental import pallas as pl
from jax.experimental.pallas import tpu as pltpu
from jax.experimental.pallas import tpu_sc as plsc

_HID = 128
_CD = 16
_NORB = 16

_NC = 2
_NS = 16
_NW = _NC * _NS
_CHUNK = 128
_RING = 5

_B_TC = 400


def _sc_gather(table, z_pad):
    b_pad = z_pad.shape[0]
    b_per_w = b_pad // _NW
    n_chunks = b_per_w // _CHUNK
    n_outer = n_chunks // _RING
    mesh = plsc.VectorSubcoreMesh(
        core_axis_name="c", subcore_axis_name="s",
        num_cores=_NC, num_subcores=_NS)

    def body(table_hbm, idx_hbm, out_hbm, idx_all, rows, gsem, ssem):
        wid = lax.axis_index("s") * _NC + lax.axis_index("c")
        base = wid * b_per_w
        pltpu.sync_copy(idx_hbm.at[pl.ds(base, b_per_w)], idx_all)

        def gath(j, b):
            return pltpu.make_async_copy(
                table_hbm.at[idx_all.at[pl.ds(j * _CHUNK, _CHUNK)]],
                rows[b], gsem[b])

        def scat(j, b):
            return pltpu.make_async_copy(
                rows[b], out_hbm.at[pl.ds(base + j * _CHUNK, _CHUNK)],
                ssem[b])

        for b in range(_RING):
            gath(b, b).start()

        def step(p, carry):
            for b in range(_RING):
                j = p * _RING + b
                gath(j, b).wait()
                scat(j, b).start()

                @pl.when(p < n_outer - 1)
                def _():
                    scat(j, b).wait()
                    gath(j + _RING, b).start()

            return carry

        lax.fori_loop(0, n_outer, step, 0)
        for b in range(_RING):
            scat(n_chunks - _RING + b, b).wait()

    f = pl.kernel(
        body,
        out_type=jax.ShapeDtypeStruct((b_pad, _HID), jnp.float32),
        mesh=mesh,
        scratch_types=[
            pltpu.VMEM((b_per_w,), jnp.int32),
            tuple(pltpu.VMEM((_CHUNK, _HID), jnp.float32)
                  for _ in range(_RING)),
            tuple(pltpu.SemaphoreType.DMA for _ in range(_RING)),
            tuple(pltpu.SemaphoreType.DMA for _ in range(_RING)),
        ],
    )
    return f(table, z_pad)


def _tc_body(cf_ref, ze_ref, w1_ref, w2_ref, out_ref):
    c = cf_ref[:]
    c = c * jax.nn.sigmoid(c)
    h = jnp.dot(c, w1_ref[:], preferred_element_type=jnp.float32)
    h = h * jax.nn.sigmoid(h)
    s = jnp.sum(h.reshape(_B_TC, _NORB, _HID), axis=1)
    out_ref[:] = ze_ref[:] + jnp.dot(
        s, w2_ref[:], preferred_element_type=jnp.float32)


def _tc_mlp(ze_pad, coeffs2, W1, W2, n):
    nblk = n // _B_TC
    return pl.pallas_call(
        _tc_body,
        grid=(nblk,),
        in_specs=[
            pl.BlockSpec((_B_TC * _NORB, _CD), lambda i: (i, 0)),
            pl.BlockSpec((_B_TC, _HID), lambda i: (i, 0)),
            pl.BlockSpec((_CD, _HID), lambda i: (0, 0)),
            pl.BlockSpec((_HID, _HID), lambda i: (0, 0)),
        ],
        out_specs=pl.BlockSpec((_B_TC, _HID), lambda i: (i, 0)),
        out_shape=jax.ShapeDtypeStruct((n, _HID), jnp.float32),
    )(coeffs2, ze_pad, W1, W2)


def kernel(z, coeffs, table, W1, W2):
    n = z.shape[0]
    granule = _NW * _CHUNK
    b_pad = ((n + granule - 1) // granule) * granule
    z_pad = jnp.pad(z.astype(jnp.int32), (0, b_pad - n))
    ze_pad = _sc_gather(table, z_pad)
    coeffs2 = coeffs.reshape(n * _NORB, _CD)
    return _tc_mlp(ze_pad, coeffs2, W1, W2, n)


# speedup vs baseline: 1.0915x; 1.0673x over previous
import functools

import jax
import jax.numpy as jnp
from jax import lax
from jax.experimental import pallas as pl
from jax.experimental.pallas import tpu as pltpu
from jax.experimental.pallas import tpu_sc as plsc

_HID = 128
_CD = 16
_NORB = 16

_NC = 2
_NS = 16
_NW = _NC * _NS
_CHUNK = 128
_RING = 5

_B_TC = 400


def _sc_gather(table, z_pad):
    b_pad = z_pad.shape[0]
    b_per_w = b_pad // _NW
    n_chunks = b_per_w // _CHUNK
    n_outer = n_chunks // _RING
    mesh = plsc.VectorSubcoreMesh(
        core_axis_name="c", subcore_axis_name="s",
        num_cores=_NC, num_subcores=_NS)

    n_rows_w = b_per_w // _CHUNK
    win = n_rows_w + 7

    def body(table_hbm, idx_hbm, out_hbm, idx_all, rows, gsem, ssem):
        wid = lax.axis_index("s") * _NC + lax.axis_index("c")
        base = wid * b_per_w
        start8 = (wid * n_rows_w) // 8 * 8
        delta = wid * n_rows_w - start8
        pltpu.sync_copy(idx_hbm.at[pl.ds(start8, win)], idx_all)

        def gath(j, b):
            return pltpu.make_async_copy(
                table_hbm.at[idx_all.at[delta + j]], rows[b], gsem[b])

        def scat(j, b):
            return pltpu.make_async_copy(
                rows[b], out_hbm.at[pl.ds(base + j * _CHUNK, _CHUNK)],
                ssem[b])

        for b in range(_RING):
            gath(b, b).start()

        def step(p, carry):
            for b in range(_RING):
                j = p * _RING + b
                gath(j, b).wait()
                scat(j, b).start()

                @pl.when(p < n_outer - 1)
                def _():
                    scat(j, b).wait()
                    gath(j + _RING, b).start()

            return carry

        lax.fori_loop(0, n_outer, step, 0)
        for b in range(_RING):
            scat(n_chunks - _RING + b, b).wait()

    f = pl.kernel(
        body,
        out_type=jax.ShapeDtypeStruct((b_pad, _HID), jnp.float32),
        mesh=mesh,
        scratch_types=[
            pltpu.VMEM((win, _CHUNK), jnp.int32),
            tuple(pltpu.VMEM((_CHUNK, _HID), jnp.float32)
                  for _ in range(_RING)),
            tuple(pltpu.SemaphoreType.DMA for _ in range(_RING)),
            tuple(pltpu.SemaphoreType.DMA for _ in range(_RING)),
        ],
    )
    return f(table, z_pad.reshape(b_pad // _CHUNK, _CHUNK))


def _silu(x):
    u = x * 0.5
    return u + u * jnp.tanh(u)


def _tc_body(cf_ref, ze_ref, w1_ref, w2_ref, out_ref):
    c = _silu(cf_ref[:])
    h = jnp.dot(c, w1_ref[:], preferred_element_type=jnp.float32)
    h = _silu(h)
    s = jnp.sum(h.reshape(_B_TC, _NORB, _HID), axis=1)
    out_ref[:] = ze_ref[:] + jnp.dot(
        s, w2_ref[:], preferred_element_type=jnp.float32)


def _tc_mlp(ze_pad, coeffs2, W1, W2, n):
    nblk = n // _B_TC
    return pl.pallas_call(
        _tc_body,
        grid=(nblk,),
        in_specs=[
            pl.BlockSpec((_B_TC * _NORB, _CD), lambda i: (i, 0)),
            pl.BlockSpec((_B_TC, _HID), lambda i: (i, 0)),
            pl.BlockSpec((_CD, _HID), lambda i: (0, 0)),
            pl.BlockSpec((_HID, _HID), lambda i: (0, 0)),
        ],
        out_specs=pl.BlockSpec((_B_TC, _HID), lambda i: (i, 0)),
        out_shape=jax.ShapeDtypeStruct((n, _HID), jnp.float32),
    )(coeffs2, ze_pad, W1, W2)


def kernel(z, coeffs, table, W1, W2):
    n = z.shape[0]
    granule = _NW * _CHUNK
    b_pad = ((n + granule - 1) // granule) * granule
    z_pad = jnp.pad(z.astype(jnp.int32), (0, b_pad - n))
    ze_pad = _sc_gather(table, z_pad)
    coeffs2 = coeffs.reshape(n * _NORB, _CD)
    return _tc_mlp(ze_pad, coeffs2, W1, W2, n)
